# Initial kernel scaffold; baseline (speedup 1.0000x reference)
import jax
import jax.numpy as jnp
from jax.experimental import pallas as pl


def kernel(x, input_lengths, w_fc1, b_fc1, w_fc2):
    raise NotImplementedError("write your pallas kernel here")


# speedup vs baseline: 13.8785x; 13.8785x over previous
import jax
import jax.numpy as jnp
from jax.experimental import pallas as pl
from jax.experimental.pallas import tpu as pltpu


def _fused_kernel(lens_ref, x_ref, w1a_ref, b1a_ref, w1b_ref, b1b_ref,
                  w2_ref, o_ref):
    C, T = x_ref.shape
    K = w2_ref.shape[0]
    pad = (K - 1) // 2
    half = C // 2
    b = pl.program_id(0)

    x = x_ref[...]
    gap = jnp.sum(x, axis=1, keepdims=True) / lens_ref[b]

    zA = jnp.dot(w1a_ref[...], gap,
                 preferred_element_type=jnp.float32) + b1a_ref[...]
    zB = jnp.dot(w1b_ref[...], gap,
                 preferred_element_type=jnp.float32) + b1b_ref[...]
    hA = jax.nn.sigmoid(zA)
    hB = jax.nn.sigmoid(zB)

    lane = jax.lax.broadcasted_iota(jnp.int32, (C, C), 1)
    scale = jnp.where(lane < half, hA, hB)

    xb = x.astype(jnp.bfloat16)
    zpad = jnp.zeros((C, pad), jnp.bfloat16)
    win = jnp.concatenate([zpad, xb, zpad], axis=1)

    acc = None
    for k in range(K):
        wk = (w2_ref[k] * scale).astype(jnp.bfloat16)
        d = jnp.dot(wk, win[:, k:k + T],
                    preferred_element_type=jnp.float32)
        acc = d if acc is None else acc + d
    o_ref[...] = acc.astype(o_ref.dtype)


def kernel(x, input_lengths, w_fc1, b_fc1, w_fc2):
    B, C, T = x.shape
    K = 5

    lens = input_lengths.astype(jnp.float32)
    w1a = w_fc1[0::2]
    w1b = w_fc1[1::2]
    b1a = b_fc1[0::2].reshape(C, 1)
    b1b = b_fc1[1::2].reshape(C, 1)
    w2 = w_fc2[:, 0].reshape(C, C, K).transpose(2, 0, 1)

    return pl.pallas_call(
        _fused_kernel,
        out_shape=jax.ShapeDtypeStruct((B, C, T), x.dtype),
        grid=(B,),
        in_specs=[
            pl.BlockSpec(memory_space=pltpu.SMEM),
            pl.BlockSpec((None, C, T), lambda b: (b, 0, 0)),
            pl.BlockSpec((C, C), lambda b: (0, 0)),
            pl.BlockSpec((C, 1), lambda b: (0, 0)),
            pl.BlockSpec((C, C), lambda b: (0, 0)),
            pl.BlockSpec((C, 1), lambda b: (0, 0)),
            pl.BlockSpec((K, C, C), lambda b: (0, 0, 0)),
        ],
        out_specs=pl.BlockSpec((None, C, T), lambda b: (b, 0, 0)),
        compiler_params=pltpu.CompilerParams(
            dimension_semantics=("parallel",),
            vmem_limit_bytes=48 * 1024 * 1024),
        name="ctxnet_fused",
    )(lens, x, w1a, b1a, w1b, b1b, w2)


# speedup vs baseline: 21.7384x; 1.5663x over previous
import jax
import jax.numpy as jnp
from jax.experimental import pallas as pl
from jax.experimental.pallas import tpu as pltpu

_K = 5


def _fused_kernel(lens_ref, x_ref, w1_ref, b1_ref, w2l_ref, o_ref, w2s_ref):
    C, T = x_ref.shape
    K = _K
    pad = (K - 1) // 2
    half = C // 2
    b = pl.program_id(0)

    @pl.when(b == 0)
    def _extract():
        w2l = w2l_ref[...].astype(jnp.bfloat16)
        rows = jax.lax.broadcasted_iota(jnp.int32, (C * K, C), 0)
        cols = jax.lax.broadcasted_iota(jnp.int32, (C * K, C), 1)
        for k in range(K):
            sel = (rows == cols * K + k).astype(jnp.bfloat16)
            w2s_ref[k] = jnp.dot(w2l, sel,
                                 preferred_element_type=jnp.float32)

    x = x_ref[...]
    ln = lens_ref[b].astype(jnp.float32)
    gap = jnp.sum(x, axis=1, keepdims=True) / ln

    z = jnp.dot(w1_ref[...], gap,
                preferred_element_type=jnp.float32)
    h = jax.nn.sigmoid(z.reshape(C, 2) + b1_ref[...])
    hA = h[:, 0:1]
    hB = h[:, 1:2]

    lane = jax.lax.broadcasted_iota(jnp.int32, (C, C), 1)
    scale = jnp.where(lane < half, hA, hB)

    xb = x.astype(jnp.bfloat16)
    zpad = jnp.zeros((C, pad), jnp.bfloat16)
    win = jnp.concatenate([zpad, xb, zpad], axis=1)

    acc = None
    for k in range(K):
        wk = (w2s_ref[k] * scale).astype(jnp.bfloat16)
        d = jnp.dot(wk, win[:, k:k + T],
                    preferred_element_type=jnp.float32)
        acc = d if acc is None else acc + d
    o_ref[...] = acc.astype(o_ref.dtype)


def kernel(x, input_lengths, w_fc1, b_fc1, w_fc2):
    B, C, T = x.shape
    K = _K

    b1r = b_fc1.reshape(C, 2)
    w2l = w_fc2.reshape(C, C * K)

    return pl.pallas_call(
        _fused_kernel,
        out_shape=jax.ShapeDtypeStruct((B, C, T), x.dtype),
        grid=(B,),
        in_specs=[
            pl.BlockSpec(memory_space=pltpu.SMEM),
            pl.BlockSpec((None, C, T), lambda b: (b, 0, 0)),
            pl.BlockSpec((2 * C, C), lambda b: (0, 0)),
            pl.BlockSpec((C, 2), lambda b: (0, 0)),
            pl.BlockSpec((C, C * K), lambda b: (0, 0)),
        ],
        out_specs=pl.BlockSpec((None, C, T), lambda b: (b, 0, 0)),
        scratch_shapes=[pltpu.VMEM((K, C, C), jnp.float32)],
        compiler_params=pltpu.CompilerParams(
            dimension_semantics=("arbitrary",),
            vmem_limit_bytes=48 * 1024 * 1024),
        name="ctxnet_fused",
    )(input_lengths, x, w_fc1, b1r, w2l)


# speedup vs baseline: 26.8547x; 1.2354x over previous
import jax
import jax.numpy as jnp
from jax.experimental import pallas as pl
from jax.experimental.pallas import tpu as pltpu

_K = 5


def _fused_kernel(lens_ref, x_ref, w1_ref, b1_ref, w2l_ref, o_ref,
                  w2s_ref, w1s_ref):
    C, T = x_ref.shape
    K = _K
    pad = (K - 1) // 2
    half = C // 2
    b = pl.program_id(0)

    @pl.when(b == 0)
    def _extract():
        w2l = w2l_ref[...].astype(jnp.bfloat16)
        rows = jax.lax.broadcasted_iota(jnp.int32, (C * K, C), 0)
        cols = jax.lax.broadcasted_iota(jnp.int32, (C * K, C), 1)
        for k in range(K):
            sel = (rows == cols * K + k).astype(jnp.bfloat16)
            w2s_ref[k] = jnp.dot(w2l, sel,
                                 preferred_element_type=jnp.float32)
        w1 = w1_ref[...].astype(jnp.bfloat16)
        rows1 = jax.lax.broadcasted_iota(jnp.int32, (C, 2 * C), 0)
        cols1 = jax.lax.broadcasted_iota(jnp.int32, (C, 2 * C), 1)
        for j in range(2):
            selj = (cols1 == 2 * rows1 + j).astype(jnp.bfloat16)
            w1s_ref[j] = jnp.dot(selj, w1,
                                 preferred_element_type=jnp.float32)

    x = x_ref[...]
    ln = lens_ref[b].astype(jnp.float32)
    gap = jnp.sum(x, axis=1, keepdims=True) / ln

    zA = jnp.dot(w1s_ref[0], gap,
                 preferred_element_type=jnp.float32) + b1_ref[:, 0:1]
    zB = jnp.dot(w1s_ref[1], gap,
                 preferred_element_type=jnp.float32) + b1_ref[:, 1:2]
    hA = jax.nn.sigmoid(zA)
    hB = jax.nn.sigmoid(zB)

    lane = jax.lax.broadcasted_iota(jnp.int32, (C, C), 1)
    scale = jnp.where(lane < half, hA, hB)

    xb = x.astype(jnp.bfloat16)
    zpad = jnp.zeros((C, pad), jnp.bfloat16)
    win = jnp.concatenate([zpad, xb, zpad], axis=1)

    acc = None
    for k in range(K):
        wk = (w2s_ref[k] * scale).astype(jnp.bfloat16)
        d = jnp.dot(wk, win[:, k:k + T],
                    preferred_element_type=jnp.float32)
        acc = d if acc is None else acc + d
    o_ref[...] = acc.astype(o_ref.dtype)


def kernel(x, input_lengths, w_fc1, b_fc1, w_fc2):
    B, C, T = x.shape
    K = _K

    b1r = b_fc1.reshape(C, 2)
    w2l = w_fc2.reshape(C, C * K)

    return pl.pallas_call(
        _fused_kernel,
        out_shape=jax.ShapeDtypeStruct((B, C, T), x.dtype),
        grid=(B,),
        in_specs=[
            pl.BlockSpec(memory_space=pltpu.SMEM),
            pl.BlockSpec((None, C, T), lambda b: (b, 0, 0)),
            pl.BlockSpec((2 * C, C), lambda b: (0, 0)),
            pl.BlockSpec((C, 2), lambda b: (0, 0)),
            pl.BlockSpec((C, C * K), lambda b: (0, 0)),
        ],
        out_specs=pl.BlockSpec((None, C, T), lambda b: (b, 0, 0)),
        scratch_shapes=[pltpu.VMEM((K, C, C), jnp.float32),
                        pltpu.VMEM((2, C, C), jnp.float32)],
        compiler_params=pltpu.CompilerParams(
            dimension_semantics=("arbitrary",),
            vmem_limit_bytes=48 * 1024 * 1024),
        name="ctxnet_fused",
    )(input_lengths, x, w_fc1, b1r, w2l)


# speedup vs baseline: 26.9472x; 1.0034x over previous
import jax
import jax.numpy as jnp
from jax.experimental import pallas as pl
from jax.experimental.pallas import tpu as pltpu

_K = 5


def _fused_kernel(lens_ref, x_ref, w1_ref, b1_ref, w2l_ref, o_ref,
                  w2s_ref, w1s_ref):
    C, T = x_ref.shape
    K = _K
    pad = (K - 1) // 2
    half = C // 2
    b = pl.program_id(0)

    @pl.when(b == 0)
    def _extract():
        w2l = w2l_ref[...].astype(jnp.bfloat16)
        rows = jax.lax.broadcasted_iota(jnp.int32, (C * K, C), 0)
        cols = jax.lax.broadcasted_iota(jnp.int32, (C * K, C), 1)
        for k in range(K):
            sel = (rows == cols * K + k).astype(jnp.bfloat16)
            w2s_ref[k] = jnp.dot(w2l, sel,
                                 preferred_element_type=jnp.float32)
        w1 = w1_ref[...].astype(jnp.bfloat16)
        rows1 = jax.lax.broadcasted_iota(jnp.int32, (C, 2 * C), 0)
        cols1 = jax.lax.broadcasted_iota(jnp.int32, (C, 2 * C), 1)
        for j in range(2):
            selj = (cols1 == 2 * rows1 + j).astype(jnp.bfloat16)
            w1s_ref[j] = jnp.dot(selj, w1,
                                 preferred_element_type=jnp.float32)

    x = x_ref[...]
    ln = lens_ref[b].astype(jnp.float32)
    gap = jnp.sum(x, axis=1, keepdims=True) / ln

    zA = jnp.dot(w1s_ref[0], gap,
                 preferred_element_type=jnp.float32) + b1_ref[:, 0:1]
    zB = jnp.dot(w1s_ref[1], gap,
                 preferred_element_type=jnp.float32) + b1_ref[:, 1:2]
    hA = jax.nn.sigmoid(zA)
    hB = jax.nn.sigmoid(zB)

    lane = jax.lax.broadcasted_iota(jnp.int32, (C, C), 1)
    scale = jnp.where(lane < half, hA, hB)

    xb = x.astype(jnp.bfloat16)
    zpad = jnp.zeros((C, pad), jnp.bfloat16)
    win = jnp.concatenate([zpad, xb, zpad], axis=1)

    acc = None
    for k in range(K):
        wk = (w2s_ref[k] * scale).astype(jnp.bfloat16)
        d = jnp.dot(wk, win[:, k:k + T],
                    preferred_element_type=jnp.float32)
        acc = d if acc is None else acc + d
    o_ref[...] = acc.astype(o_ref.dtype)


def kernel(x, input_lengths, w_fc1, b_fc1, w_fc2):
    B, C, T = x.shape
    K = _K

    b1r = b_fc1.reshape(C, 2)
    w2l = w_fc2[:, 0].reshape(C, C * K)

    return pl.pallas_call(
        _fused_kernel,
        out_shape=jax.ShapeDtypeStruct((B, C, T), x.dtype),
        grid=(B,),
        in_specs=[
            pl.BlockSpec(memory_space=pltpu.SMEM),
            pl.BlockSpec((None, C, T), lambda b: (b, 0, 0)),
            pl.BlockSpec((2 * C, C), lambda b: (0, 0)),
            pl.BlockSpec((C, 2), lambda b: (0, 0)),
            pl.BlockSpec((C, C * K), lambda b: (0, 0)),
        ],
        out_specs=pl.BlockSpec((None, C, T), lambda b: (b, 0, 0)),
        scratch_shapes=[pltpu.VMEM((K, C, C), jnp.float32),
                        pltpu.VMEM((2, C, C), jnp.float32)],
        compiler_params=pltpu.CompilerParams(
            dimension_semantics=("arbitrary",),
            vmem_limit_bytes=48 * 1024 * 1024),
        name="ctxnet_fused",
    )(input_lengths, x, w_fc1, b1r, w2l)


# speedup vs baseline: 33.2205x; 1.2328x over previous
import jax
import jax.numpy as jnp
from jax.experimental import pallas as pl
from jax.experimental.pallas import tpu as pltpu

_K = 5


def _fused_kernel(lens_ref, x_ref, w1_ref, b1_ref, w2l_ref, o_ref,
                  w2s_ref, w1s_ref):
    C, T = x_ref.shape
    K = _K
    pad = (K - 1) // 2
    half = C // 2
    b = pl.program_id(0)

    @pl.when(b == 0)
    def _extract():
        w2l = w2l_ref[...].reshape(C, C * K).astype(jnp.bfloat16)
        rows = jax.lax.broadcasted_iota(jnp.int32, (C * K, C), 0)
        cols = jax.lax.broadcasted_iota(jnp.int32, (C * K, C), 1)
        for k in range(K):
            sel = (rows == cols * K + k).astype(jnp.bfloat16)
            w2s_ref[k] = jnp.dot(w2l, sel,
                                 preferred_element_type=jnp.float32)
        w1 = w1_ref[...].astype(jnp.bfloat16)
        rows1 = jax.lax.broadcasted_iota(jnp.int32, (C, 2 * C), 0)
        cols1 = jax.lax.broadcasted_iota(jnp.int32, (C, 2 * C), 1)
        for j in range(2):
            selj = (cols1 == 2 * rows1 + j).astype(jnp.bfloat16)
            w1s_ref[j] = jnp.dot(selj, w1,
                                 preferred_element_type=jnp.float32)

    x = x_ref[...]
    ln = lens_ref[b].astype(jnp.float32)
    gap = jnp.sum(x, axis=1, keepdims=True) / ln

    zA = jnp.dot(w1s_ref[0], gap,
                 preferred_element_type=jnp.float32) + b1_ref[:, 0:1]
    zB = jnp.dot(w1s_ref[1], gap,
                 preferred_element_type=jnp.float32) + b1_ref[:, 1:2]
    hA = jax.nn.sigmoid(zA)
    hB = jax.nn.sigmoid(zB)

    lane = jax.lax.broadcasted_iota(jnp.int32, (C, C), 1)
    scale = jnp.where(lane < half, hA, hB)

    xb = x.astype(jnp.bfloat16)
    zpad = jnp.zeros((C, pad), jnp.bfloat16)
    win = jnp.concatenate([zpad, xb, zpad], axis=1)

    acc = None
    for k in range(K):
        wk = (w2s_ref[k] * scale).astype(jnp.bfloat16)
        d = jnp.dot(wk, win[:, k:k + T],
                    preferred_element_type=jnp.float32)
        acc = d if acc is None else acc + d
    o_ref[...] = acc.astype(o_ref.dtype)


def kernel(x, input_lengths, w_fc1, b_fc1, w_fc2):
    B, C, T = x.shape
    K = _K

    b1r = b_fc1.reshape(C, 2)
    w2l = w_fc2.reshape(2 * C * K, C // 2)

    return pl.pallas_call(
        _fused_kernel,
        out_shape=jax.ShapeDtypeStruct((B, C, T), x.dtype),
        grid=(B,),
        in_specs=[
            pl.BlockSpec(memory_space=pltpu.SMEM),
            pl.BlockSpec((None, C, T), lambda b: (b, 0, 0)),
            pl.BlockSpec((2 * C, C), lambda b: (0, 0)),
            pl.BlockSpec((C, 2), lambda b: (0, 0)),
            pl.BlockSpec((2 * C * K, C // 2), lambda b: (0, 0)),
        ],
        out_specs=pl.BlockSpec((None, C, T), lambda b: (b, 0, 0)),
        scratch_shapes=[pltpu.VMEM((K, C, C), jnp.float32),
                        pltpu.VMEM((2, C, C), jnp.float32)],
        compiler_params=pltpu.CompilerParams(
            dimension_semantics=("arbitrary",),
            vmem_limit_bytes=48 * 1024 * 1024),
        name="ctxnet_fused",
    )(input_lengths, x, w_fc1, b1r, w2l)
